# Initial kernel scaffold; baseline (speedup 1.0000x reference)
import jax
import jax.numpy as jnp
from jax.experimental import pallas as pl


def kernel(sequence, segment_label, token_table, segment_table, pe):
    raise NotImplementedError("write your pallas kernel here")


# speedup vs baseline: 1.8608x; 1.8608x over previous
import functools

import jax
import jax.numpy as jnp
from jax import lax
from jax.experimental import pallas as pl
from jax.experimental.pallas import tpu as pltpu
from jax.experimental.pallas import tpu_sc as plsc

_B, _L, _E = 1024, 200, 64
_N = _B * _L
_NC, _NS = 2, 16
_NW = _NC * _NS
_ROWS_W = _N // _NW
_CH = 128
_NCH = _ROWS_W // _CH

_mesh = plsc.VectorSubcoreMesh(core_axis_name="c", subcore_axis_name="s")


@functools.partial(
    pl.kernel,
    mesh=_mesh,
    out_type=jax.ShapeDtypeStruct((_N, _E), jnp.float32),
    scratch_types=[
        pltpu.VMEM((_NCH, _CH), jnp.int32),
        pltpu.VMEM((_NCH, _CH), jnp.int32),
        pltpu.VMEM((_L, _E), jnp.float32),
        pltpu.VMEM((_CH, _E), jnp.float32),
        pltpu.VMEM((_CH, _E), jnp.float32),
        pltpu.VMEM((_CH, _E), jnp.float32),
        pltpu.SemaphoreType.DMA,
        pltpu.SemaphoreType.DMA,
    ],
    compiler_params=pltpu.CompilerParams(use_tc_tiling_on_sc=False),
)
def _emb_kernel(seq_hbm, seglab_hbm, tok_tab, seg_tab, pe_hbm, out_hbm,
                tokidx, segidx, pe_v, tokb, segb, outb, sem_t, sem_s):
    wid = lax.axis_index("s") * _NC + lax.axis_index("c")
    base = wid * _ROWS_W

    pltpu.sync_copy(seq_hbm.at[wid], tokidx)
    pltpu.sync_copy(seglab_hbm.at[wid], segidx)
    pltpu.sync_copy(pe_hbm, pe_v)

    def chunk_body(g, carry):
        pltpu.async_copy(tok_tab.at[tokidx.at[g]], tokb, sem_t).wait()
        pltpu.async_copy(seg_tab.at[segidx.at[g]], segb, sem_s).wait()

        def row_body(r, c2):
            pos = lax.rem(g * _CH + r, _L)
            for p2 in range(_E // 16):
                sl = pl.ds(p2 * 16, 16)
                outb[r, sl] = (tokb[r, sl] + pe_v[pos, sl]) + segb[r, sl]
            return c2

        lax.fori_loop(0, _CH, row_body, 0)
        pltpu.sync_copy(outb, out_hbm.at[pl.ds(base + g * _CH, _CH)])
        return carry

    lax.fori_loop(0, _NCH, chunk_body, 0)


def kernel(sequence, segment_label, token_table, segment_table, pe):
    seq = sequence.reshape(_NW, _NCH, _CH).astype(jnp.int32)
    seg = segment_label.reshape(_NW, _NCH, _CH).astype(jnp.int32)
    pe_l = pe[0, :_L, :].astype(jnp.float32)
    out = _emb_kernel(seq, seg, token_table, segment_table, pe_l)
    return out.reshape(_B, _L, _E)


# speedup vs baseline: 1.8872x; 1.0142x over previous
import functools

import jax
import jax.numpy as jnp
from jax import lax
from jax.experimental import pallas as pl
from jax.experimental.pallas import tpu as pltpu
from jax.experimental.pallas import tpu_sc as plsc

_B, _L, _E = 1024, 200, 64
_N = _B * _L
_NC, _NS = 2, 16
_NW = _NC * _NS
_ROWS_W = _N // _NW
_CH = 128
_NCH = _ROWS_W // _CH

_mesh = plsc.VectorSubcoreMesh(core_axis_name="c", subcore_axis_name="s")


@functools.partial(
    pl.kernel,
    mesh=_mesh,
    out_type=jax.ShapeDtypeStruct((_N, _E), jnp.float32),
    scratch_types=[
        pltpu.VMEM((_NCH, _CH), jnp.int32),
        pltpu.VMEM((_NCH, _CH), jnp.int32),
        pltpu.VMEM((_L, _E), jnp.float32),
        [pltpu.VMEM((_CH, _E), jnp.float32) for _ in range(2)],
        [pltpu.VMEM((_CH, _E), jnp.float32) for _ in range(2)],
        [pltpu.VMEM((_CH, _E), jnp.float32) for _ in range(2)],
        [pltpu.SemaphoreType.DMA for _ in range(6)],
    ],
    compiler_params=pltpu.CompilerParams(use_tc_tiling_on_sc=False),
)
def _emb_kernel(seq_hbm, seglab_hbm, tok_tab, seg_tab, pe_hbm, out_hbm,
                tokidx, segidx, pe_v, tokb, segb, outb, sems):
    wid = lax.axis_index("s") * _NC + lax.axis_index("c")
    base = wid * _ROWS_W
    sem_gt, sem_gs, sem_sc = sems[0:2], sems[2:4], sems[4:6]

    pltpu.sync_copy(seq_hbm.at[wid], tokidx)
    pltpu.sync_copy(seglab_hbm.at[wid], segidx)
    pltpu.sync_copy(pe_hbm, pe_v)

    def gathers(t, b):
        return (pltpu.make_async_copy(tok_tab.at[tokidx.at[t]], tokb[b], sem_gt[b]),
                pltpu.make_async_copy(seg_tab.at[segidx.at[t]], segb[b], sem_gs[b]))

    def scatter(t, b):
        return pltpu.make_async_copy(outb[b], out_hbm.at[pl.ds(base + t * _CH, _CH)],
                                     sem_sc[b])

    def compute(t, b):
        tb, sb, ob = tokb[b], segb[b], outb[b]

        def row_body(r, c2):
            pos = lax.rem(t * _CH + r, _L)
            for p2 in range(_E // 16):
                sl = pl.ds(p2 * 16, 16)
                ob[r, sl] = (tb[r, sl] + pe_v[pos, sl]) + sb[r, sl]
            return c2

        lax.fori_loop(0, _CH, row_body, 0)

    for b in range(2):
        for h in gathers(b, b):
            h.start()
    for b in range(2):
        for h in gathers(b, b):
            h.wait()
        compute(b, b)
        scatter(b, b).start()
        for h in gathers(b + 2, b):
            h.start()

    def steady(u, carry):
        for b in range(2):
            t = 2 * u + b
            for h in gathers(t, b):
                h.wait()
            scatter(t - 2, b).wait()
            compute(t, b)
            scatter(t, b).start()

            @pl.when(t + 2 < _NCH)
            def _():
                for h in gathers(t + 2, b):
                    h.start()
        return carry

    lax.fori_loop(1, _NCH // 2, steady, 0)

    for b in range(2):
        scatter(_NCH - 2 + b, b).wait()


def kernel(sequence, segment_label, token_table, segment_table, pe):
    seq = sequence.reshape(_NW, _NCH, _CH).astype(jnp.int32)
    seg = segment_label.reshape(_NW, _NCH, _CH).astype(jnp.int32)
    pe_l = pe[0, :_L, :].astype(jnp.float32)
    out = _emb_kernel(seq, seg, token_table, segment_table, pe_l)
    return out.reshape(_B, _L, _E)
